# baseline (speedup 1.0000x reference)
import jax
import jax.numpy as jnp
from jax import lax
from jax.experimental import pallas as pl
from jax.experimental.pallas import tpu as pltpu
from jax.experimental.pallas import tpu_sc as plsc

_N = 10000
_E = 320000
_DIN = 128
_DH = 16
_DO = 5

_NC = 2
_NS = 16
_NW = _NC * _NS
_CH = 128
_NCHUNK = 80
_EP = _NW * _NCHUNK * _CH
_RN = 10240
_RPT = _RN // _NS

_mesh = plsc.VectorSubcoreMesh(core_axis_name="c", subcore_axis_name="s")
_sc_params = pltpu.CompilerParams(use_tc_tiling_on_sc=False)


_RCH = _RPT * _DH // _CH


def _deg_body(dst3, zrow, ridx, degr, didx, ones, rxi, rep, acc,
              gsem, ssem_a, ssem_b):
    c = lax.axis_index("c")
    s = lax.axis_index("s")
    wid = c * _NS + s
    base = s * _RPT
    for k in range(_CH // 16):
        ones[pl.ds(k * 16, 16)] = jnp.full((16,), 1.0, jnp.float32)
    pltpu.sync_copy(zrow, acc.at[pl.ds(base, _RPT)])
    pltpu.async_copy(dst3.at[wid], didx, gsem).wait()
    pltpu.async_copy(ridx.at[s], rxi, gsem).wait()
    plsc.subcore_barrier()

    def fire(k, carry):
        pltpu.async_copy(ones, acc.at[didx.at[2 * k]], ssem_a, add=True)
        pltpu.async_copy(ones, acc.at[didx.at[2 * k + 1]], ssem_b, add=True)
        return carry

    lax.fori_loop(0, _NCHUNK // 2, fire, 0)

    def drain(k, carry):
        pltpu.make_async_copy(ones, acc.at[didx.at[0]], ssem_a).wait()
        pltpu.make_async_copy(ones, acc.at[didx.at[0]], ssem_b).wait()
        return carry

    lax.fori_loop(0, _NCHUNK // 2, drain, 0)
    plsc.subcore_barrier()

    def rfire(k, carry):
        pltpu.async_copy(acc.at[rxi.at[2 * k]], rep.at[2 * k], ssem_a)
        pltpu.async_copy(acc.at[rxi.at[2 * k + 1]], rep.at[2 * k + 1], ssem_b)
        return carry

    lax.fori_loop(0, _RCH // 2, rfire, 0)

    def rdrain(k, carry):
        pltpu.make_async_copy(acc.at[rxi.at[0]], rep.at[0], ssem_a).wait()
        pltpu.make_async_copy(acc.at[rxi.at[0]], rep.at[0], ssem_b).wait()
        return carry

    lax.fori_loop(0, _RCH // 2, rdrain, 0)
    pltpu.sync_copy(rep, degr.at[c, pl.ds(s * _RCH, _RCH)])


_deg_call = pl.kernel(
    _deg_body,
    out_type=jax.ShapeDtypeStruct((_NC, _RN * _DH // _CH, _CH), jnp.float32),
    mesh=_mesh,
    scratch_types=[
        pltpu.VMEM((_NCHUNK, _CH), jnp.int32),
        pltpu.VMEM((_CH,), jnp.float32),
        pltpu.VMEM((_RCH, _CH), jnp.int32),
        pltpu.VMEM((_RCH, _CH), jnp.float32),
        pltpu.VMEM_SHARED((_RN,), jnp.float32),
        pltpu.SemaphoreType.DMA,
        pltpu.SemaphoreType.DMA,
        pltpu.SemaphoreType.DMA,
    ],
    compiler_params=_sc_params,
)


_NBUF = 8
_LAG = _NBUF // 2


def _agg_body(hs, src3, dst3, zrows, accp, sidx, didx, buf, acc, hs_spm,
              gs0, gs1, gs2, gs3, gs4, gs5, gs6, gs7,
              ss0, ss1, ss2, ss3, ss4, ss5, ss6, ss7, lsem):
    c = lax.axis_index("c")
    s = lax.axis_index("s")
    wid = c * _NS + s
    base = s * _RPT
    gsem = [gs0, gs1, gs2, gs3, gs4, gs5, gs6, gs7]
    ssem = [ss0, ss1, ss2, ss3, ss4, ss5, ss6, ss7]
    pltpu.async_copy(src3.at[wid], sidx, lsem)
    pltpu.async_copy(dst3.at[wid], didx, lsem)
    pltpu.sync_copy(hs.at[pl.ds(base, _RPT)], hs_spm.at[pl.ds(base, _RPT)])
    pltpu.sync_copy(zrows, acc.at[pl.ds(base, _RPT)])
    pltpu.make_async_copy(src3.at[wid], sidx, lsem).wait()
    pltpu.make_async_copy(dst3.at[wid], didx, lsem).wait()
    plsc.subcore_barrier()

    def gather(j, b):
        pltpu.async_copy(hs_spm.at[sidx.at[j]], buf.at[b], gsem[b])

    def wait_gather(b):
        pltpu.make_async_copy(hs.at[sidx.at[0]], buf.at[b], gsem[b]).wait()

    def scatter(j, b):
        pltpu.async_copy(buf.at[b], acc.at[didx.at[j]], ssem[b], add=True)

    def wait_scatter(b):
        pltpu.make_async_copy(buf.at[b], acc.at[didx.at[0]], ssem[b]).wait()

    for b in range(_LAG):
        gather(b, b)

    def step(k, carry):
        for b in range(_NBUF):
            j = k * _NBUF + b
            wait_gather(b)
            scatter(j, b)
            nb = (b + _LAG) % _NBUF
            if b >= _LAG:
                wait_scatter(nb)
                gather(jnp.minimum(j + _LAG, _NCHUNK - 1), nb)
            else:
                @pl.when(k > 0)
                def _():
                    wait_scatter(nb)
                gather(jnp.minimum(j + _LAG, _NCHUNK - 1), nb)
        return carry

    lax.fori_loop(0, _NCHUNK // _NBUF, step, 0)
    for b in range(_LAG):
        wait_scatter(b + _LAG)
        wait_gather(b)
    plsc.subcore_barrier()
    pltpu.sync_copy(acc.at[pl.ds(base, _RPT)],
                    accp.at[c, pl.ds(base, _RPT)])


_agg_call = pl.kernel(
    _agg_body,
    out_type=jax.ShapeDtypeStruct((_NC, _RN, _DH), jnp.float32),
    mesh=_mesh,
    scratch_types=[
        pltpu.VMEM((_NCHUNK, _CH), jnp.int32),
        pltpu.VMEM((_NCHUNK, _CH), jnp.int32),
        pltpu.VMEM((_NBUF, _CH, _DH), jnp.float32),
        pltpu.VMEM_SHARED((_RN, _DH), jnp.float32),
        pltpu.VMEM_SHARED((_RN, _DH), jnp.float32),
    ] + [pltpu.SemaphoreType.DMA] * (2 * _NBUF + 1),
    compiler_params=_sc_params,
)


def _agg2_body(accp1, hs1, disb, b1, src3, dst3, zrows, accp, hs2out,
               sidx, didx, buf, acc, hs_spm, va, vb, vh, vd, bias,
               gs0, gs1, gs2, gs3, gs4, gs5, gs6, gs7,
               ss0, ss1, ss2, ss3, ss4, ss5, ss6, ss7, lsem):
    c = lax.axis_index("c")
    s = lax.axis_index("s")
    wid = c * _NS + s
    base = s * _RPT
    gsem = [gs0, gs1, gs2, gs3, gs4, gs5, gs6, gs7]
    ssem = [ss0, ss1, ss2, ss3, ss4, ss5, ss6, ss7]
    pltpu.async_copy(src3.at[wid], sidx, lsem)
    pltpu.async_copy(dst3.at[wid], didx, lsem)
    pltpu.sync_copy(zrows, acc.at[pl.ds(base, _RPT)])
    pltpu.sync_copy(accp1.at[0, pl.ds(base, _RPT)], va)
    pltpu.sync_copy(accp1.at[1, pl.ds(base, _RPT)], vb)
    pltpu.sync_copy(hs1.at[pl.ds(base, _RPT)], vh)
    pltpu.sync_copy(disb.at[pl.ds(base, _RPT)], vd)
    pltpu.sync_copy(b1, bias)

    def rowmath(r, carry):
        d = vd[r, :]
        u = (va[r, :] + vb[r, :] + vh[r, :]) * d
        vh[r, :] = jnp.maximum(u + bias[...], 0.0) * d
        return carry

    lax.fori_loop(0, _RPT, rowmath, 0)
    pltpu.sync_copy(vh, hs_spm.at[pl.ds(base, _RPT)])

    @pl.when(c == 0)
    def _():
        pltpu.sync_copy(vh, hs2out.at[pl.ds(base, _RPT)])

    pltpu.make_async_copy(src3.at[wid], sidx, lsem).wait()
    pltpu.make_async_copy(dst3.at[wid], didx, lsem).wait()
    plsc.subcore_barrier()

    def gather(j, b):
        pltpu.async_copy(hs_spm.at[sidx.at[j]], buf.at[b], gsem[b])

    def wait_gather(b):
        pltpu.make_async_copy(hs_spm.at[sidx.at[0]], buf.at[b], gsem[b]).wait()

    def scatter(j, b):
        pltpu.async_copy(buf.at[b], acc.at[didx.at[j]], ssem[b], add=True)

    def wait_scatter(b):
        pltpu.make_async_copy(buf.at[b], acc.at[didx.at[0]], ssem[b]).wait()

    for b in range(_LAG):
        gather(b, b)

    def step(k, carry):
        for b in range(_NBUF):
            j = k * _NBUF + b
            wait_gather(b)
            scatter(j, b)
            nb = (b + _LAG) % _NBUF
            if b >= _LAG:
                wait_scatter(nb)
                gather(jnp.minimum(j + _LAG, _NCHUNK - 1), nb)
            else:
                @pl.when(k > 0)
                def _():
                    wait_scatter(nb)
                gather(jnp.minimum(j + _LAG, _NCHUNK - 1), nb)
        return carry

    lax.fori_loop(0, _NCHUNK // _NBUF, step, 0)
    for b in range(_LAG):
        wait_scatter(b + _LAG)
        wait_gather(b)
    plsc.subcore_barrier()
    pltpu.sync_copy(acc.at[pl.ds(base, _RPT)],
                    accp.at[c, pl.ds(base, _RPT)])


_agg2_call = pl.kernel(
    _agg2_body,
    out_type=[jax.ShapeDtypeStruct((_NC, _RN, _DH), jnp.float32),
              jax.ShapeDtypeStruct((_RN, _DH), jnp.float32)],
    mesh=_mesh,
    scratch_types=[
        pltpu.VMEM((_NCHUNK, _CH), jnp.int32),
        pltpu.VMEM((_NCHUNK, _CH), jnp.int32),
        pltpu.VMEM((_NBUF, _CH, _DH), jnp.float32),
        pltpu.VMEM_SHARED((_RN, _DH), jnp.float32),
        pltpu.VMEM_SHARED((_RN, _DH), jnp.float32),
        pltpu.VMEM((_RPT, _DH), jnp.float32),
        pltpu.VMEM((_RPT, _DH), jnp.float32),
        pltpu.VMEM((_RPT, _DH), jnp.float32),
        pltpu.VMEM((_RPT, _DH), jnp.float32),
        pltpu.VMEM((_DH,), jnp.float32),
    ] + [pltpu.SemaphoreType.DMA] * (2 * _NBUF + 1),
    compiler_params=_sc_params,
)


_BR = 2048
_RV = _RN * _DH // 128
_BV = _BR * _DH // 128
_grid = (_RN // _BR,)

_view_spec = pl.BlockSpec((_BV, 128), lambda i: (i, 0))
_degv_spec = pl.BlockSpec((_NC, _BV, 128), lambda i: (0, i, 0))
_NEG = -1e30


def _stage_a(x2_ref, w1b_ref, degv_ref, hsv_ref, disb_ref):
    disb = lax.rsqrt(degv_ref[0] + degv_ref[1] + 1.0)
    hv = jnp.dot(x2_ref[...], w1b_ref[...], preferred_element_type=jnp.float32)
    hsv_ref[...] = hv * disb
    disb_ref[...] = disb


def _stage_b(accv_ref, hsv_ref, disb_ref, b1v_ref, hs2v_ref):
    disb = disb_ref[...]
    u = (accv_ref[0] + accv_ref[1] + hsv_ref[...]) * disb
    h2 = jnp.maximum(u + b1v_ref[...], 0.0)
    hs2v_ref[...] = h2 * disb


def _shift_left(a, k):
    return jnp.concatenate(
        [a[:, k:], jnp.full((a.shape[0], k), _NEG, jnp.float32)], axis=1)


def _stage_c(accv_ref, hs2v_ref, disb_ref, w2b_ref, b2b_ref, out_ref):
    u2v = (accv_ref[0] + accv_ref[1] + hs2v_ref[...]) * disb_ref[...]
    o = jnp.dot(u2v, w2b_ref[...], preferred_element_type=jnp.float32)
    o = o + b2b_ref[...]
    s = jnp.maximum(o, _shift_left(o, 1))
    s = jnp.maximum(s, _shift_left(s, 2))
    s = jnp.maximum(s, _shift_left(s, 4))
    r_ = lax.broadcasted_iota(jnp.int32, (64, 64), 0)
    c_ = lax.broadcasted_iota(jnp.int32, (64, 64), 1)
    gpick = jnp.where(r_ == (c_ // 8) * 8, 1.0, 0.0)
    gsum = jnp.where(r_ // 8 == c_ // 8, 1.0, 0.0)
    m = jnp.dot(s, gpick, preferred_element_type=jnp.float32)
    e = jnp.exp(o - m)
    se = jnp.dot(e, gsum, preferred_element_type=jnp.float32)
    out_ref[...] = (o - m) - jnp.log(se)


def _call_a(x2, W1b, degv):
    return pl.pallas_call(
        _stage_a,
        grid=_grid,
        in_specs=[pl.BlockSpec((_BV, 8 * _DIN), lambda i: (i, 0)),
                  pl.BlockSpec((8 * _DIN, 128), lambda i: (0, 0)),
                  _degv_spec],
        out_specs=[_view_spec, _view_spec],
        out_shape=[jax.ShapeDtypeStruct((_RV, 128), jnp.float32),
                   jax.ShapeDtypeStruct((_RV, 128), jnp.float32)],
    )(x2, W1b, degv)


def _call_b(accv, hsv, disb, b1v):
    return pl.pallas_call(
        _stage_b,
        grid=_grid,
        in_specs=[_degv_spec, _view_spec, _view_spec,
                  pl.BlockSpec((128,), lambda i: (0,))],
        out_specs=_view_spec,
        out_shape=jax.ShapeDtypeStruct((_RV, 128), jnp.float32),
    )(accv, hsv, disb, b1v)


def _call_c(accv, hs2v, disb, W2b, b2b):
    return pl.pallas_call(
        _stage_c,
        grid=_grid,
        in_specs=[_degv_spec, _view_spec, _view_spec,
                  pl.BlockSpec((128, 64), lambda i: (0, 0)),
                  pl.BlockSpec((64,), lambda i: (0,))],
        out_specs=pl.BlockSpec((_BV, 64), lambda i: (i, 0)),
        out_shape=jax.ShapeDtypeStruct((_RV, 64), jnp.float32),
    )(accv, hs2v, disb, W2b, b2b)


def kernel(x, edge_index, W1, b1, W2, b2):
    src = edge_index[0]
    dst = edge_index[1]
    pad = _EP - _E
    pad_src = jnp.zeros((pad,), jnp.int32)
    pad_dst = _N + (jnp.arange(pad, dtype=jnp.int32) % 8)
    src3 = jnp.concatenate([src, pad_src]).reshape(_NW, _NCHUNK, _CH)
    dst3 = jnp.concatenate([dst, pad_dst]).reshape(_NW, _NCHUNK, _CH)
    x2 = jnp.pad(x, ((0, _RN - _N), (0, 0))).reshape(_RV, 8 * _DIN)
    zrow = jnp.zeros((_RPT,), jnp.float32)
    zrows = jnp.zeros((_RPT, _DH), jnp.float32)
    ridx = (jnp.arange(_NS, dtype=jnp.int32)[:, None, None] * _RPT
            + jnp.arange(_RCH, dtype=jnp.int32)[None, :, None] * 8
            + jnp.arange(_CH, dtype=jnp.int32)[None, None, :] // _DH)
    eye8 = jnp.eye(8, dtype=jnp.float32)
    W1b = jnp.kron(eye8, W1)
    W2b = jnp.kron(eye8, jnp.pad(W2, ((0, 0), (0, 3))))
    b2b = jnp.tile(jnp.concatenate([b2, jnp.full((3,), _NEG, jnp.float32)]), 8)

    degr = _deg_call(dst3, zrow, ridx)
    hsv, disb = _call_a(x2, W1b, degr)
    accp1 = _agg_call(hsv.reshape(_RN, _DH), src3, dst3, zrows)
    accp2, hs2 = _agg2_call(accp1, hsv.reshape(_RN, _DH),
                            disb.reshape(_RN, _DH), b1, src3, dst3, zrows)
    o2 = _call_c(accp2.reshape(_NC, _RV, 128), hs2.reshape(_RV, 128),
                 disb, W2b, b2b)
    return o2.reshape(_RN, 8)[:_N, :_DO]


# speedup vs baseline: 1.0653x; 1.0653x over previous
import jax
import jax.numpy as jnp
from jax import lax
from jax.experimental import pallas as pl
from jax.experimental.pallas import tpu as pltpu
from jax.experimental.pallas import tpu_sc as plsc

_N = 10000
_E = 320000
_DIN = 128
_DH = 16
_DO = 5

_NC = 2
_NS = 16
_NW = _NC * _NS
_CH = 128
_NCHUNK = 80
_EP = _NW * _NCHUNK * _CH
_RN = 10240
_RPT = _RN // _NS

_mesh = plsc.VectorSubcoreMesh(core_axis_name="c", subcore_axis_name="s")
_sc_params = pltpu.CompilerParams(use_tc_tiling_on_sc=False)


_RCH = _RPT * _DH // _CH


def _deg_body(dst3, zrow, ridx, degr, didx, ones, rxi, rep, acc,
              gsem, ssem_a, ssem_b):
    c = lax.axis_index("c")
    s = lax.axis_index("s")
    wid = c * _NS + s
    base = s * _RPT
    for k in range(_CH // 16):
        ones[pl.ds(k * 16, 16)] = jnp.full((16,), 1.0, jnp.float32)
    pltpu.sync_copy(zrow, acc.at[pl.ds(base, _RPT)])
    pltpu.async_copy(dst3.at[wid], didx, gsem).wait()
    pltpu.async_copy(ridx.at[s], rxi, gsem).wait()
    plsc.subcore_barrier()

    def fire(k, carry):
        pltpu.async_copy(ones, acc.at[didx.at[2 * k]], ssem_a, add=True)
        pltpu.async_copy(ones, acc.at[didx.at[2 * k + 1]], ssem_b, add=True)
        return carry

    lax.fori_loop(0, _NCHUNK // 2, fire, 0)

    def drain(k, carry):
        pltpu.make_async_copy(ones, acc.at[didx.at[0]], ssem_a).wait()
        pltpu.make_async_copy(ones, acc.at[didx.at[0]], ssem_b).wait()
        return carry

    lax.fori_loop(0, _NCHUNK // 2, drain, 0)
    plsc.subcore_barrier()

    def rfire(k, carry):
        pltpu.async_copy(acc.at[rxi.at[2 * k]], rep.at[2 * k], ssem_a)
        pltpu.async_copy(acc.at[rxi.at[2 * k + 1]], rep.at[2 * k + 1], ssem_b)
        return carry

    lax.fori_loop(0, _RCH // 2, rfire, 0)

    def rdrain(k, carry):
        pltpu.make_async_copy(acc.at[rxi.at[0]], rep.at[0], ssem_a).wait()
        pltpu.make_async_copy(acc.at[rxi.at[0]], rep.at[0], ssem_b).wait()
        return carry

    lax.fori_loop(0, _RCH // 2, rdrain, 0)
    pltpu.sync_copy(rep, degr.at[c, pl.ds(s * _RCH, _RCH)])


_deg_call = pl.kernel(
    _deg_body,
    out_type=jax.ShapeDtypeStruct((_NC, _RN * _DH // _CH, _CH), jnp.float32),
    mesh=_mesh,
    scratch_types=[
        pltpu.VMEM((_NCHUNK, _CH), jnp.int32),
        pltpu.VMEM((_CH,), jnp.float32),
        pltpu.VMEM((_RCH, _CH), jnp.int32),
        pltpu.VMEM((_RCH, _CH), jnp.float32),
        pltpu.VMEM_SHARED((_RN,), jnp.float32),
        pltpu.SemaphoreType.DMA,
        pltpu.SemaphoreType.DMA,
        pltpu.SemaphoreType.DMA,
    ],
    compiler_params=_sc_params,
)


_NBUF = 8
_LAG = _NBUF // 2


def _agg_body(hs, src3, dst3, zrows, accp, sidx, didx, buf, acc, hs_spm,
              gs0, gs1, gs2, gs3, gs4, gs5, gs6, gs7,
              ss0, ss1, ss2, ss3, ss4, ss5, ss6, ss7, lsem):
    c = lax.axis_index("c")
    s = lax.axis_index("s")
    wid = c * _NS + s
    base = s * _RPT
    gsem = [gs0, gs1, gs2, gs3, gs4, gs5, gs6, gs7]
    ssem = [ss0, ss1, ss2, ss3, ss4, ss5, ss6, ss7]
    pltpu.async_copy(src3.at[wid], sidx, lsem)
    pltpu.async_copy(dst3.at[wid], didx, lsem)
    pltpu.sync_copy(hs.at[pl.ds(base, _RPT)], hs_spm.at[pl.ds(base, _RPT)])
    pltpu.sync_copy(zrows, acc.at[pl.ds(base, _RPT)])
    pltpu.make_async_copy(src3.at[wid], sidx, lsem).wait()
    pltpu.make_async_copy(dst3.at[wid], didx, lsem).wait()
    plsc.subcore_barrier()

    def gather(j, b):
        pltpu.async_copy(hs_spm.at[sidx.at[j]], buf.at[b], gsem[b])

    def wait_gather(b):
        pltpu.make_async_copy(hs.at[sidx.at[0]], buf.at[b], gsem[b]).wait()

    def scatter(j, b):
        pltpu.async_copy(buf.at[b], acc.at[didx.at[j]], ssem[b], add=True)

    def wait_scatter(b):
        pltpu.make_async_copy(buf.at[b], acc.at[didx.at[0]], ssem[b]).wait()

    for b in range(_LAG):
        gather(b, b)

    def step(k, carry):
        for b in range(_NBUF):
            j = k * _NBUF + b
            wait_gather(b)
            scatter(j, b)
            nb = (b + _LAG) % _NBUF
            if b >= _LAG:
                wait_scatter(nb)
                gather(jnp.minimum(j + _LAG, _NCHUNK - 1), nb)
            else:
                @pl.when(k > 0)
                def _():
                    wait_scatter(nb)
                gather(jnp.minimum(j + _LAG, _NCHUNK - 1), nb)
        return carry

    lax.fori_loop(0, _NCHUNK // _NBUF, step, 0)
    for b in range(_LAG):
        wait_scatter(b + _LAG)
        wait_gather(b)
    plsc.subcore_barrier()
    pltpu.sync_copy(acc.at[pl.ds(base, _RPT)],
                    accp.at[c, pl.ds(base, _RPT)])


_agg_call = pl.kernel(
    _agg_body,
    out_type=jax.ShapeDtypeStruct((_NC, _RN, _DH), jnp.float32),
    mesh=_mesh,
    scratch_types=[
        pltpu.VMEM((_NCHUNK, _CH), jnp.int32),
        pltpu.VMEM((_NCHUNK, _CH), jnp.int32),
        pltpu.VMEM((_NBUF, _CH, _DH), jnp.float32),
        pltpu.VMEM_SHARED((_RN, _DH), jnp.float32),
        pltpu.VMEM_SHARED((_RN, _DH), jnp.float32),
    ] + [pltpu.SemaphoreType.DMA] * (2 * _NBUF + 1),
    compiler_params=_sc_params,
)


_BR = 2048
_RV = _RN * _DH // 128
_BV = _BR * _DH // 128
_grid = (_RN // _BR,)

_view_spec = pl.BlockSpec((_BV, 128), lambda i: (i, 0))
_degv_spec = pl.BlockSpec((_NC, _BV, 128), lambda i: (0, i, 0))
_NEG = -1e30


def _stage_a(x2_ref, w1b_ref, degv_ref, hsv_ref, disb_ref):
    disb = lax.rsqrt(degv_ref[0] + degv_ref[1] + 1.0)
    hv = jnp.dot(x2_ref[...], w1b_ref[...], preferred_element_type=jnp.float32)
    hsv_ref[...] = hv * disb
    disb_ref[...] = disb


def _stage_b(accv_ref, hsv_ref, disb_ref, b1v_ref, hs2v_ref):
    disb = disb_ref[...]
    u = (accv_ref[0] + accv_ref[1] + hsv_ref[...]) * disb
    h2 = jnp.maximum(u + b1v_ref[...], 0.0)
    hs2v_ref[...] = h2 * disb


def _shift_left(a, k):
    return jnp.concatenate(
        [a[:, k:], jnp.full((a.shape[0], k), _NEG, jnp.float32)], axis=1)


def _stage_c(accv_ref, hs2v_ref, disb_ref, w2b_ref, b2b_ref, out_ref):
    u2v = (accv_ref[0] + accv_ref[1] + hs2v_ref[...]) * disb_ref[...]
    o = jnp.dot(u2v, w2b_ref[...], preferred_element_type=jnp.float32)
    o = o + b2b_ref[...]
    s = jnp.maximum(o, _shift_left(o, 1))
    s = jnp.maximum(s, _shift_left(s, 2))
    s = jnp.maximum(s, _shift_left(s, 4))
    r_ = lax.broadcasted_iota(jnp.int32, (64, 64), 0)
    c_ = lax.broadcasted_iota(jnp.int32, (64, 64), 1)
    gpick = jnp.where(r_ == (c_ // 8) * 8, 1.0, 0.0)
    gsum = jnp.where(r_ // 8 == c_ // 8, 1.0, 0.0)
    m = jnp.dot(s, gpick, preferred_element_type=jnp.float32)
    e = jnp.exp(o - m)
    se = jnp.dot(e, gsum, preferred_element_type=jnp.float32)
    out_ref[...] = (o - m) - jnp.log(se)


def _call_a(x2, W1b, degv):
    return pl.pallas_call(
        _stage_a,
        grid=_grid,
        in_specs=[pl.BlockSpec((_BV, 8 * _DIN), lambda i: (i, 0)),
                  pl.BlockSpec((8 * _DIN, 128), lambda i: (0, 0)),
                  _degv_spec],
        out_specs=[_view_spec, _view_spec],
        out_shape=[jax.ShapeDtypeStruct((_RV, 128), jnp.float32),
                   jax.ShapeDtypeStruct((_RV, 128), jnp.float32)],
    )(x2, W1b, degv)


def _call_b(accv, hsv, disb, b1v):
    return pl.pallas_call(
        _stage_b,
        grid=_grid,
        in_specs=[_degv_spec, _view_spec, _view_spec,
                  pl.BlockSpec((128,), lambda i: (0,))],
        out_specs=_view_spec,
        out_shape=jax.ShapeDtypeStruct((_RV, 128), jnp.float32),
    )(accv, hsv, disb, b1v)


def _call_c(accv, hs2v, disb, W2b, b2b):
    return pl.pallas_call(
        _stage_c,
        grid=_grid,
        in_specs=[_degv_spec, _view_spec, _view_spec,
                  pl.BlockSpec((128, 64), lambda i: (0, 0)),
                  pl.BlockSpec((64,), lambda i: (0,))],
        out_specs=pl.BlockSpec((_BV, 64), lambda i: (i, 0)),
        out_shape=jax.ShapeDtypeStruct((_RV, 64), jnp.float32),
    )(accv, hs2v, disb, W2b, b2b)


def kernel(x, edge_index, W1, b1, W2, b2):
    src = edge_index[0]
    dst = edge_index[1]
    pad = _EP - _E
    pad_src = jnp.zeros((pad,), jnp.int32)
    pad_dst = _N + (jnp.arange(pad, dtype=jnp.int32) % 8)
    src3 = jnp.concatenate([src, pad_src]).reshape(_NW, _NCHUNK, _CH)
    dst3 = jnp.concatenate([dst, pad_dst]).reshape(_NW, _NCHUNK, _CH)
    x2 = jnp.pad(x, ((0, _RN - _N), (0, 0))).reshape(_RV, 8 * _DIN)
    zrow = jnp.zeros((_RPT,), jnp.float32)
    zrows = jnp.zeros((_RPT, _DH), jnp.float32)
    ridx = (jnp.arange(_NS, dtype=jnp.int32)[:, None, None] * _RPT
            + jnp.arange(_RCH, dtype=jnp.int32)[None, :, None] * 8
            + jnp.arange(_CH, dtype=jnp.int32)[None, None, :] // _DH)
    eye8 = jnp.eye(8, dtype=jnp.float32)
    W1b = jnp.kron(eye8, W1)
    W2b = jnp.kron(eye8, jnp.pad(W2, ((0, 0), (0, 3))))
    b2b = jnp.tile(jnp.concatenate([b2, jnp.full((3,), _NEG, jnp.float32)]), 8)

    degr = _deg_call(dst3, zrow, ridx)
    hsv, disb = _call_a(x2, W1b, degr)
    accp1 = _agg_call(hsv.reshape(_RN, _DH), src3, dst3, zrows)
    hs2v = _call_b(accp1.reshape(_NC, _RV, 128), hsv, disb, jnp.tile(b1, 8))
    accp2 = _agg_call(hs2v.reshape(_RN, _DH), src3, dst3, zrows)
    o2 = _call_c(accp2.reshape(_NC, _RV, 128), hs2v, disb, W2b, b2b)
    return o2.reshape(_RN, 8)[:_N, :_DO]
